# baseline (speedup 1.0000x reference)
import functools

import jax
import jax.numpy as jnp
from jax import lax
from jax.experimental import pallas as pl
from jax.experimental.pallas import tpu as pltpu
from jax.experimental.pallas import tpu_sc as plsc

SEQ_LEN = 200
EMBED = 64
VOCAB = 100000
CHUNK = 128
NUM_CORES = 2
NUM_SUBCORES = 16
NUM_WORKERS = NUM_CORES * NUM_SUBCORES
NBUF = 4


def _sc_body(idx_hbm, tok_hbm, pos_hbm, out_hbm, pos2_v, idx_v, rows, gsems, ssems):
    wid = lax.axis_index("s") * NUM_CORES + lax.axis_index("c")
    n_chunks = idx_hbm.shape[0] // NUM_WORKERS
    chunk0 = wid * n_chunks

    pltpu.sync_copy(idx_hbm.at[pl.ds(chunk0, n_chunks)], idx_v)
    pltpu.sync_copy(pos_hbm, pos2_v.at[pl.ds(0, SEQ_LEN)])
    pltpu.sync_copy(pos_hbm, pos2_v.at[pl.ds(SEQ_LEN, SEQ_LEN)])

    def fire_gather(c, b):
        pltpu.async_copy(tok_hbm.at[idx_v.at[c]], rows.at[b], gsems.at[b])

    def drain_gather(b):
        pltpu.make_async_copy(
            tok_hbm.at[pl.ds(0, CHUNK)], rows.at[b], gsems.at[b]
        ).wait()

    def fire_scatter(c, b):
        pltpu.async_copy(
            rows.at[b],
            out_hbm.at[pl.ds((chunk0 + c) * CHUNK, CHUNK)],
            ssems.at[b],
        )

    def drain_scatter(b):
        pltpu.make_async_copy(
            rows.at[b], out_hbm.at[pl.ds(0, CHUNK)], ssems.at[b]
        ).wait()

    def add_pos(c, b):
        s0 = lax.rem(c * CHUNK, SEQ_LEN)

        def row_body(r, carry):
            for e in range(EMBED // 16):
                sl = pl.ds(e * 16, 16)
                rows[b, r, sl] = rows[b, r, sl] + pos2_v[s0 + r, sl]
            return carry

        lax.fori_loop(0, CHUNK, row_body, 0)

    for b in range(NBUF):
        fire_gather(b, b)

    n_iters = n_chunks // NBUF

    def body(i, carry):
        for j in range(NBUF):
            c = i * NBUF + j
            drain_gather(j)

            @pl.when(i > 0)
            def _():
                drain_scatter(j)

            add_pos(c, j)
            fire_scatter(c, j)

            @pl.when(i < n_iters - 1)
            def _():
                fire_gather(c + NBUF, j)

        return carry

    lax.fori_loop(0, n_iters, body, 0)

    for b in range(NBUF):
        drain_scatter(b)


def kernel(inputs, token_table, position_table):
    batch = inputs.shape[0]
    idx = inputs.astype(jnp.int32).reshape(batch * SEQ_LEN // CHUNK, CHUNK)

    mesh = plsc.VectorSubcoreMesh(core_axis_name="c", subcore_axis_name="s")
    k = functools.partial(
        pl.kernel,
        out_type=jax.ShapeDtypeStruct((batch * SEQ_LEN, EMBED), jnp.float32),
        mesh=mesh,
        compiler_params=pltpu.CompilerParams(use_tc_tiling_on_sc=False),
        scratch_types=[
            pltpu.VMEM((2 * SEQ_LEN, EMBED), jnp.float32),
            pltpu.VMEM((batch * SEQ_LEN // CHUNK // NUM_WORKERS, CHUNK), jnp.int32),
            pltpu.VMEM((NBUF, CHUNK, EMBED), jnp.float32),
            pltpu.SemaphoreType.DMA((NBUF,)),
            pltpu.SemaphoreType.DMA((NBUF,)),
        ],
    )(_sc_body)
    out = k(idx, token_table, position_table)
    return out.reshape(batch, SEQ_LEN, EMBED)


# speedup vs baseline: 1.4055x; 1.4055x over previous
import functools

import jax
import jax.numpy as jnp
from jax import lax
from jax.experimental import pallas as pl
from jax.experimental.pallas import tpu as pltpu
from jax.experimental.pallas import tpu_sc as plsc

SEQ_LEN = 200
EMBED = 64
VOCAB = 100000
CHUNK = 64
NUM_CORES = 2
NUM_SUBCORES = 16
NUM_WORKERS = NUM_CORES * NUM_SUBCORES
NBUF = 2
BLK = 8


def _sc_body(
    idx_hbm, tok_hbm, pos_hbm, out_hbm, posp_v, idx_v, gbuf, rows, gsems, ssems, isem
):
    wid = lax.axis_index("s") * NUM_CORES + lax.axis_index("c")
    n_idx_rows = idx_hbm.shape[0] // NUM_WORKERS
    n_chunks = 2 * n_idx_rows
    irow0 = wid * n_idx_rows
    chunk0 = wid * n_chunks

    pltpu.sync_copy(pos_hbm, posp_v)
    pltpu.sync_copy(idx_hbm.at[pl.ds(irow0, BLK)], idx_v.at[0])

    def fire_gather(c, b):
        irow = c // 2
        slot = (irow // BLK) % 2
        row = irow % BLK
        half = (c % 2) * CHUNK
        for k in range(2):
            pltpu.async_copy(
                tok_hbm.at[idx_v.at[slot, row, pl.ds(half + k * 32, 32)]],
                gbuf.at[b, pl.ds(k * 32, 32)],
                gsems.at[b],
            )

    def drain_gather(b):
        pltpu.make_async_copy(
            tok_hbm.at[pl.ds(0, CHUNK)], gbuf.at[b], gsems.at[b]
        ).wait()

    def fire_scatter(c, b):
        pltpu.async_copy(
            rows.at[b], out_hbm.at[pl.ds((chunk0 + c) * CHUNK, CHUNK)], ssems.at[b]
        )

    def drain_scatter(b):
        pltpu.make_async_copy(
            rows.at[b], out_hbm.at[pl.ds(0, CHUNK)], ssems.at[b]
        ).wait()

    def add_pos(c, b):
        s0 = lax.rem(c * CHUNK, SEQ_LEN)
        p0 = s0 // 2
        n1 = lax.min(CHUNK // 2, (SEQ_LEN - s0) // 2)

        def make_body(pbase):
            def pair_body(p, carry):
                for h in range(2):
                    for e in range(EMBED // 16):
                        sl = pl.ds(e * 16, 16)
                        psl = pl.ds(h * EMBED + e * 16, 16)
                        r = 2 * p + h
                        rows[b, r, sl] = gbuf[b, r, sl] + posp_v[p + pbase, psl]
                return carry

            return pair_body

        lax.fori_loop(0, n1, make_body(p0), 0)
        lax.fori_loop(n1, CHUNK // 2, make_body(p0 - SEQ_LEN // 2), 0)

    for b in range(NBUF):
        fire_gather(b, b)

    n_iters = n_chunks // NBUF

    def body(i, carry):
        for j in range(NBUF):
            c = i * NBUF + j
            drain_gather(j)

            @pl.when(i > 0)
            def _():
                drain_scatter(j)

            add_pos(c, j)
            fire_scatter(c, j)

            if j == 0:
                @pl.when(lax.rem(c, 2 * BLK) == 0)
                def _():
                    @pl.when(c + 2 * BLK < n_chunks)
                    def _():
                        blk = c // (2 * BLK) + 1
                        pltpu.async_copy(
                            idx_hbm.at[pl.ds(irow0 + blk * BLK, BLK)],
                            idx_v.at[blk % 2],
                            isem,
                        )

                @pl.when(lax.rem(c + 2, 2 * BLK) == 0)
                def _():
                    @pl.when(c + 2 < n_chunks)
                    def _():
                        pltpu.make_async_copy(
                            idx_hbm.at[pl.ds(0, BLK)], idx_v.at[0], isem
                        ).wait()

            @pl.when(i < n_iters - 1)
            def _():
                fire_gather(c + NBUF, j)

        return carry

    lax.fori_loop(0, n_iters, body, 0)

    for b in range(NBUF):
        drain_scatter(b)


def kernel(inputs, token_table, position_table):
    batch = inputs.shape[0]
    idx = inputs.astype(jnp.int32).reshape(batch * SEQ_LEN // 128, 128)
    tok = jnp.pad(token_table, ((0, 0), (0, EMBED)))
    posp = position_table.reshape(SEQ_LEN // 2, 2 * EMBED)

    mesh = plsc.VectorSubcoreMesh(core_axis_name="c", subcore_axis_name="s")
    k = functools.partial(
        pl.kernel,
        out_type=jax.ShapeDtypeStruct((batch * SEQ_LEN, EMBED), jnp.float32),
        mesh=mesh,
        scratch_types=[
            pltpu.VMEM((SEQ_LEN // 2, 2 * EMBED), jnp.float32),
            pltpu.VMEM((2, BLK, 128), jnp.int32),
            pltpu.VMEM((NBUF, CHUNK, 2 * EMBED), jnp.float32),
            pltpu.VMEM((NBUF, CHUNK, EMBED), jnp.float32),
            pltpu.SemaphoreType.DMA((NBUF,)),
            pltpu.SemaphoreType.DMA((NBUF,)),
            pltpu.SemaphoreType.DMA,
        ],
    )(_sc_body)
    out = k(idx, tok, posp)
    return out.reshape(batch, SEQ_LEN, EMBED)


# speedup vs baseline: 1.8503x; 1.3164x over previous
import functools

import jax
import jax.numpy as jnp
from jax import lax
from jax.experimental import pallas as pl
from jax.experimental.pallas import tpu as pltpu
from jax.experimental.pallas import tpu_sc as plsc

SEQ_LEN = 200
EMBED = 64
VOCAB = 100000
CHUNK = 64
NUM_CORES = 2
NUM_SUBCORES = 16
NUM_WORKERS = NUM_CORES * NUM_SUBCORES
NBUF = 2
BLK = 8


def _sc_body(
    idx_hbm, tok_hbm, pos_hbm, out_hbm, posp_v, idx_v, gbuf, rows, gsems, ssems, isem
):
    wid = lax.axis_index("s") * NUM_CORES + lax.axis_index("c")
    n_idx_rows = idx_hbm.shape[0] // NUM_WORKERS
    n_chunks = 2 * n_idx_rows
    irow0 = wid * n_idx_rows
    chunk0 = wid * n_chunks

    pltpu.sync_copy(pos_hbm, posp_v.at[pl.ds(0, SEQ_LEN // 2)])
    pltpu.sync_copy(pos_hbm, posp_v.at[pl.ds(SEQ_LEN // 2, SEQ_LEN // 2)])
    pltpu.sync_copy(idx_hbm.at[pl.ds(irow0, BLK)], idx_v.at[0])

    def fire_gather(c, b):
        irow = c // 2
        slot = (irow // BLK) % 2
        row = irow % BLK
        half = (c % 2) * CHUNK
        for k in range(2):
            pltpu.async_copy(
                tok_hbm.at[idx_v.at[slot, row, pl.ds(half + k * 32, 32)]],
                gbuf.at[b, pl.ds(k * 32, 32)],
                gsems.at[b],
            )

    def drain_gather(b):
        pltpu.make_async_copy(
            tok_hbm.at[pl.ds(0, CHUNK)], gbuf.at[b], gsems.at[b]
        ).wait()

    def fire_scatter(c, b):
        pltpu.async_copy(
            rows.at[b], out_hbm.at[pl.ds((chunk0 + c) * CHUNK, CHUNK)], ssems.at[b]
        )

    def drain_scatter(b):
        pltpu.make_async_copy(
            rows.at[b], out_hbm.at[pl.ds(0, CHUNK)], ssems.at[b]
        ).wait()

    def add_pos(c, b):
        p0 = lax.rem(c * (CHUNK // 2), SEQ_LEN // 2)

        @plsc.parallel_loop(0, CHUNK // 2, unroll=4)
        def pair_body(p):
            for h in range(2):
                for e in range(EMBED // 16):
                    sl = pl.ds(e * 16, 16)
                    psl = pl.ds(h * EMBED + e * 16, 16)
                    r = 2 * p + h
                    rows[b, r, sl] = gbuf[b, r, sl] + posp_v[p0 + p, psl]

    for b in range(NBUF):
        fire_gather(b, b)

    n_iters = n_chunks // NBUF

    def body(i, carry):
        for j in range(NBUF):
            c = i * NBUF + j
            drain_gather(j)

            @pl.when(i > 0)
            def _():
                drain_scatter(j)

            add_pos(c, j)
            fire_scatter(c, j)

            if j == 0:
                @pl.when(lax.rem(c, 2 * BLK) == 0)
                def _():
                    @pl.when(c + 2 * BLK < n_chunks)
                    def _():
                        blk = c // (2 * BLK) + 1
                        pltpu.async_copy(
                            idx_hbm.at[pl.ds(irow0 + blk * BLK, BLK)],
                            idx_v.at[blk % 2],
                            isem,
                        )

                @pl.when(lax.rem(c + 2, 2 * BLK) == 0)
                def _():
                    @pl.when(c + 2 < n_chunks)
                    def _():
                        pltpu.make_async_copy(
                            idx_hbm.at[pl.ds(0, BLK)], idx_v.at[0], isem
                        ).wait()

            @pl.when(i < n_iters - 1)
            def _():
                fire_gather(c + NBUF, j)

        return carry

    lax.fori_loop(0, n_iters, body, 0)

    for b in range(NBUF):
        drain_scatter(b)


def kernel(inputs, token_table, position_table):
    batch = inputs.shape[0]
    idx = inputs.astype(jnp.int32).reshape(batch * SEQ_LEN // 128, 128)
    tok = jnp.pad(token_table, ((0, 0), (0, EMBED)))
    posp = position_table.reshape(SEQ_LEN // 2, 2 * EMBED)

    mesh = plsc.VectorSubcoreMesh(core_axis_name="c", subcore_axis_name="s")
    k = functools.partial(
        pl.kernel,
        out_type=jax.ShapeDtypeStruct((batch * SEQ_LEN, EMBED), jnp.float32),
        mesh=mesh,
        scratch_types=[
            pltpu.VMEM((SEQ_LEN, 2 * EMBED), jnp.float32),
            pltpu.VMEM((2, BLK, 128), jnp.int32),
            pltpu.VMEM((NBUF, CHUNK, 2 * EMBED), jnp.float32),
            pltpu.VMEM((NBUF, CHUNK, EMBED), jnp.float32),
            pltpu.SemaphoreType.DMA((NBUF,)),
            pltpu.SemaphoreType.DMA((NBUF,)),
            pltpu.SemaphoreType.DMA,
        ],
    )(_sc_body)
    out = k(idx, tok, posp)
    return out.reshape(batch, SEQ_LEN, EMBED)
